# Initial kernel scaffold; baseline (speedup 1.0000x reference)
import jax
import jax.numpy as jnp
from jax.experimental import pallas as pl


def kernel(x, edge_index, W1, b1, W2, b2):
    raise NotImplementedError("write your pallas kernel here")


# speedup vs baseline: 5.3424x; 5.3424x over previous
import functools

import jax
import jax.numpy as jnp
from jax import lax
from jax.experimental import pallas as pl
from jax.experimental.pallas import tpu as pltpu
from jax.experimental.pallas import tpu_sc as plsc

N = 10000
E = 320000
D1 = 128
D2 = 64
NC = 2
NS = 16
NW = NC * NS
EPW = E // NW
K = 80
C = EPW // K
W8 = 624
TAIL = N - NS * W8

_MESH = plsc.VectorSubcoreMesh(
    core_axis_name="c", subcore_axis_name="s", num_cores=NC, num_subcores=NS
)


def _make_deg():
    def body(src_hbm, dst_hbm, ones_hbm, zeros_hbm, out_hbm,
             idx_v, ones_v, tmp_v, dsrc_sh, ddst_sh):
        c = lax.axis_index("c")
        s = lax.axis_index("s")
        wid = s * NC + c

        @pl.when(s == 0)
        def _zero():
            pltpu.sync_copy(zeros_hbm, tmp_v)
            pltpu.sync_copy(tmp_v, dsrc_sh)
            pltpu.sync_copy(tmp_v, ddst_sh)

        pltpu.sync_copy(ones_hbm, ones_v)
        plsc.subcore_barrier()

        pltpu.sync_copy(src_hbm.at[wid], idx_v)

        def sbody(j, carry):
            pltpu.sync_copy(ones_v, dsrc_sh.at[idx_v.at[j]], add=True)
            return carry

        lax.fori_loop(0, C, sbody, 0)

        pltpu.sync_copy(dst_hbm.at[wid], idx_v)

        def dbody(j, carry):
            pltpu.sync_copy(ones_v, ddst_sh.at[idx_v.at[j]], add=True)
            return carry

        lax.fori_loop(0, C, dbody, 0)

        plsc.subcore_barrier()

        @pl.when(s == 0)
        def _out():
            pltpu.sync_copy(dsrc_sh, tmp_v)
            pltpu.sync_copy(tmp_v, out_hbm.at[pl.ds((c * 2 + 0) * N, N)])
            pltpu.sync_copy(ddst_sh, tmp_v)
            pltpu.sync_copy(tmp_v, out_hbm.at[pl.ds((c * 2 + 1) * N, N)])

    return pl.kernel(
        body,
        out_type=jax.ShapeDtypeStruct((NC * 2 * N,), jnp.float32),
        mesh=_MESH,
        scratch_types=[
            pltpu.VMEM((C, K), jnp.int32),
            pltpu.VMEM((K,), jnp.float32),
            pltpu.VMEM((N,), jnp.float32),
            pltpu.VMEM_SHARED((N,), jnp.float32),
            pltpu.VMEM_SHARED((N,), jnp.float32),
        ],
    )


def _make_agg(Dh):
    EPT = E // NS
    CC = EPT // K

    def body(y_hbm, src_hbm, dst_hbm, zeros_hbm, out_hbm,
             srcv, dstv, rows, tmp, acc_sh, sem):
        c = lax.axis_index("c")
        s = lax.axis_index("s")

        pltpu.sync_copy(zeros_hbm, tmp)
        pltpu.sync_copy(tmp, acc_sh.at[pl.ds(s * W8, W8)])

        @pl.when(s == 0)
        def _zero_tail():
            pltpu.sync_copy(zeros_hbm.at[pl.ds(0, TAIL)], rows.at[pl.ds(0, TAIL)])
            pltpu.sync_copy(rows.at[pl.ds(0, TAIL)], acc_sh.at[pl.ds(NS * W8, TAIL)])

        pltpu.sync_copy(src_hbm.at[s], srcv)
        pltpu.sync_copy(dst_hbm.at[s], dstv)
        plsc.subcore_barrier()

        yc = y_hbm.at[c]

        def step(j, carry):
            pltpu.async_copy(yc.at[srcv.at[j]], rows, sem).wait()
            pltpu.sync_copy(rows, acc_sh.at[dstv.at[j]], add=True)
            return carry

        lax.fori_loop(0, CC, step, 0)

        plsc.subcore_barrier()
        pltpu.sync_copy(acc_sh.at[pl.ds(s * W8, W8)], tmp)
        pltpu.sync_copy(tmp, out_hbm.at[c, pl.ds(s * W8, W8)])

        @pl.when(s == 0)
        def _out_tail():
            pltpu.sync_copy(acc_sh.at[pl.ds(NS * W8, TAIL)], rows.at[pl.ds(0, TAIL)])
            pltpu.sync_copy(rows.at[pl.ds(0, TAIL)], out_hbm.at[c, pl.ds(NS * W8, TAIL)])

    return pl.kernel(
        body,
        out_type=jax.ShapeDtypeStruct((NC, N, Dh), jnp.float32),
        mesh=_MESH,
        scratch_types=[
            pltpu.VMEM((CC, K), jnp.int32),
            pltpu.VMEM((CC, K), jnp.int32),
            pltpu.VMEM((K, Dh), jnp.float32),
            pltpu.VMEM((W8, Dh), jnp.float32),
            pltpu.VMEM_SHARED((N, Dh), jnp.float32),
            pltpu.SemaphoreType.DMA,
        ],
        compiler_params=pltpu.CompilerParams(use_tc_tiling_on_sc=False),
    )


DH1 = D1 // NC
DH2 = D2 // NC

_deg_call = _make_deg()
_agg1_call = _make_agg(DH1)
_agg2_call = _make_agg(DH2)


BN = 1000


def _prep_body(x_ref, ds_ref, dd_ref, w1_ref, y_ref, ns_ref, nd_ref):
    d_s = ds_ref[...]
    d_d = dd_ref[...]
    ns = lax.rsqrt(jnp.maximum(d_s[:, 0:1] + d_s[:, 1:2], 1.0))
    nd = lax.rsqrt(jnp.maximum(d_d[:, 0:1] + d_d[:, 1:2], 1.0))
    ns_ref[...] = ns
    nd_ref[...] = nd
    y = jnp.dot(x_ref[...] * ns, w1_ref[...],
                preferred_element_type=jnp.float32)
    y_ref[...] = jnp.stack([y[:, :DH1], y[:, DH1:]])


def _mid_body(p_ref, ns_ref, nd_ref, b1_ref, w2_ref, y2_ref):
    p = p_ref[...]
    agg = jnp.concatenate([p[0], p[1]], axis=1)
    h = jnp.maximum(agg * nd_ref[...] + b1_ref[...], 0.0)
    y2 = jnp.dot(h * ns_ref[...], w2_ref[...],
                 preferred_element_type=jnp.float32)
    y2_ref[...] = jnp.stack([y2[:, :DH2], y2[:, DH2:]])


def _fin_body(p_ref, nd_ref, b2_ref, o_ref):
    p = p_ref[...]
    o_ref[...] = jnp.concatenate([p[0], p[1]], axis=1) * nd_ref[...] + b2_ref[...]


def _prep(x, dsrcT, ddstT, W1):
    return pl.pallas_call(
        _prep_body,
        grid=(N // BN,),
        in_specs=[
            pl.BlockSpec((BN, D1), lambda i: (i, 0)),
            pl.BlockSpec((BN, 2), lambda i: (i, 0)),
            pl.BlockSpec((BN, 2), lambda i: (i, 0)),
            pl.BlockSpec((D1, D1), lambda i: (0, 0)),
        ],
        out_specs=[
            pl.BlockSpec((NC, BN, DH1), lambda i: (0, i, 0)),
            pl.BlockSpec((BN, 1), lambda i: (i, 0)),
            pl.BlockSpec((BN, 1), lambda i: (i, 0)),
        ],
        out_shape=[
            jax.ShapeDtypeStruct((NC, N, DH1), jnp.float32),
            jax.ShapeDtypeStruct((N, 1), jnp.float32),
            jax.ShapeDtypeStruct((N, 1), jnp.float32),
        ],
    )(x, dsrcT, ddstT, W1)


def _mid(agg1p, nsT, ndT, b1, W2):
    return pl.pallas_call(
        _mid_body,
        grid=(N // BN,),
        in_specs=[
            pl.BlockSpec((NC, BN, DH1), lambda i: (0, i, 0)),
            pl.BlockSpec((BN, 1), lambda i: (i, 0)),
            pl.BlockSpec((BN, 1), lambda i: (i, 0)),
            pl.BlockSpec((1, D1), lambda i: (0, 0)),
            pl.BlockSpec((D1, D2), lambda i: (0, 0)),
        ],
        out_specs=pl.BlockSpec((NC, BN, DH2), lambda i: (0, i, 0)),
        out_shape=jax.ShapeDtypeStruct((NC, N, DH2), jnp.float32),
    )(agg1p, nsT, ndT, b1, W2)


def _fin(agg2p, ndT, b2):
    return pl.pallas_call(
        _fin_body,
        grid=(N // BN,),
        in_specs=[
            pl.BlockSpec((NC, BN, DH2), lambda i: (0, i, 0)),
            pl.BlockSpec((BN, 1), lambda i: (i, 0)),
            pl.BlockSpec((1, D2), lambda i: (0, 0)),
        ],
        out_specs=pl.BlockSpec((BN, D2), lambda i: (i, 0)),
        out_shape=jax.ShapeDtypeStruct((N, D2), jnp.float32),
    )(agg2p, ndT, b2)


def kernel(x, edge_index, W1, b1, W2, b2):
    src32 = edge_index[0].astype(jnp.int32)
    dst32 = edge_index[1].astype(jnp.int32)
    src_d = src32.reshape(NW, C, K)
    dst_d = dst32.reshape(NW, C, K)
    src_a = src32.reshape(NS, E // NS // K, K)
    dst_a = dst32.reshape(NS, E // NS // K, K)
    ones_k = jnp.ones((K,), jnp.float32)
    zeros_n = jnp.zeros((N,), jnp.float32)
    zeros_r1 = jnp.zeros((W8, DH1), jnp.float32)
    zeros_r2 = jnp.zeros((W8, DH2), jnp.float32)

    degs = _deg_call(src_d, dst_d, ones_k, zeros_n).reshape(NC, 2, N)
    dsrcT = degs[:, 0, :].T
    ddstT = degs[:, 1, :].T

    y1, nsT, ndT = _prep(x, dsrcT, ddstT, W1)
    agg1h = _agg1_call(y1, src_a, dst_a, zeros_r1)
    y2 = _mid(agg1h, nsT, ndT, b1.reshape(1, D1), W2)
    agg2h = _agg2_call(y2, src_a, dst_a, zeros_r2)
    return _fin(agg2h, ndT, b2.reshape(1, D2))


# speedup vs baseline: 9.6437x; 1.8051x over previous
import functools

import jax
import jax.numpy as jnp
from jax import lax
from jax.experimental import pallas as pl
from jax.experimental.pallas import tpu as pltpu
from jax.experimental.pallas import tpu_sc as plsc

N = 10000
E = 320000
D1 = 128
D2 = 64
NC = 2
NS = 16
NW = NC * NS
EPW = E // NW
K = 80
C = EPW // K
W8 = 624
TAIL = N - NS * W8
KA = 125
CA = (E // NS) // KA

_MESH = plsc.VectorSubcoreMesh(
    core_axis_name="c", subcore_axis_name="s", num_cores=NC, num_subcores=NS
)


def _make_deg():
    def body(src_hbm, dst_hbm, ones_hbm, zeros_hbm, out_hbm,
             idx_v, ones_v, tmp_v, dsrc_sh, ddst_sh):
        c = lax.axis_index("c")
        s = lax.axis_index("s")
        wid = s * NC + c

        @pl.when(s == 0)
        def _zero():
            pltpu.sync_copy(zeros_hbm, tmp_v)
            pltpu.sync_copy(tmp_v, dsrc_sh)
            pltpu.sync_copy(tmp_v, ddst_sh)

        pltpu.sync_copy(ones_hbm, ones_v)
        plsc.subcore_barrier()

        pltpu.sync_copy(src_hbm.at[wid], idx_v)

        def sbody(j, carry):
            pltpu.sync_copy(ones_v, dsrc_sh.at[idx_v.at[j]], add=True)
            return carry

        lax.fori_loop(0, C, sbody, 0)

        pltpu.sync_copy(dst_hbm.at[wid], idx_v)

        def dbody(j, carry):
            pltpu.sync_copy(ones_v, ddst_sh.at[idx_v.at[j]], add=True)
            return carry

        lax.fori_loop(0, C, dbody, 0)

        plsc.subcore_barrier()

        @pl.when(s == 0)
        def _out():
            pltpu.sync_copy(dsrc_sh, tmp_v)
            pltpu.sync_copy(tmp_v, out_hbm.at[pl.ds((c * 2 + 0) * N, N)])
            pltpu.sync_copy(ddst_sh, tmp_v)
            pltpu.sync_copy(tmp_v, out_hbm.at[pl.ds((c * 2 + 1) * N, N)])

    return pl.kernel(
        body,
        out_type=jax.ShapeDtypeStruct((NC * 2 * N,), jnp.float32),
        mesh=_MESH,
        scratch_types=[
            pltpu.VMEM((C, K), jnp.int32),
            pltpu.VMEM((K,), jnp.float32),
            pltpu.VMEM((N,), jnp.float32),
            pltpu.VMEM_SHARED((N,), jnp.float32),
            pltpu.VMEM_SHARED((N,), jnp.float32),
        ],
    )


def _make_agg(Dh):
    CC = CA
    NB = 2

    def body(y_hbm, src_hbm, dst_hbm, out_hbm,
             srcv, dstv, rows0, rows1, acc_sh, sem0):
        c = lax.axis_index("c")
        s = lax.axis_index("s")
        rows = (rows0, rows1)
        sems = (sem0, sem0)

        def zbody(i, carry):
            for k_ in range(Dh // 16):
                rows0[i, pl.ds(16 * k_, 16)] = jnp.zeros((16,), jnp.float32)
            return carry

        lax.fori_loop(0, KA, zbody, 0)
        for k_ in range(4):
            pltpu.sync_copy(rows0, acc_sh.at[pl.ds(s * W8 + 125 * k_, 125)])
        pltpu.sync_copy(rows0.at[pl.ds(0, 124)],
                        acc_sh.at[pl.ds(s * W8 + 500, 124)])

        @pl.when(s == 0)
        def _zero_tail():
            pltpu.sync_copy(rows0.at[pl.ds(0, TAIL)],
                            acc_sh.at[pl.ds(NS * W8, TAIL)])

        pltpu.sync_copy(src_hbm.at[s], srcv)
        pltpu.sync_copy(dst_hbm.at[s], dstv)
        plsc.subcore_barrier()

        yc = y_hbm.at[c]

        for b in range(NB):
            pltpu.async_copy(yc.at[srcv.at[b]], rows[b], sems[b])

        def step(jj, carry):
            for b in range(NB):
                j = jj * NB + b
                pltpu.make_async_copy(yc.at[srcv.at[j]], rows[b], sems[b]).wait()
                pltpu.sync_copy(rows[b], acc_sh.at[dstv.at[j]], add=True)

                @pl.when(j + NB < CC)
                def _refill():
                    pltpu.async_copy(yc.at[srcv.at[j + NB]], rows[b], sems[b])
            return carry

        lax.fori_loop(0, CC // NB, step, 0)

        plsc.subcore_barrier()
        for k_ in range(4):
            pltpu.sync_copy(acc_sh.at[pl.ds(s * W8 + 125 * k_, 125)], rows0)
            pltpu.sync_copy(rows0, out_hbm.at[c, pl.ds(s * W8 + 125 * k_, 125)])
        pltpu.sync_copy(acc_sh.at[pl.ds(s * W8 + 500, 124)],
                        rows0.at[pl.ds(0, 124)])
        pltpu.sync_copy(rows0.at[pl.ds(0, 124)],
                        out_hbm.at[c, pl.ds(s * W8 + 500, 124)])

        @pl.when(s == 0)
        def _out_tail():
            pltpu.sync_copy(acc_sh.at[pl.ds(NS * W8, TAIL)], rows1.at[pl.ds(0, TAIL)])
            pltpu.sync_copy(rows1.at[pl.ds(0, TAIL)], out_hbm.at[c, pl.ds(NS * W8, TAIL)])

    return pl.kernel(
        body,
        out_type=jax.ShapeDtypeStruct((NC, N, Dh), jnp.float32),
        mesh=_MESH,
        scratch_types=[
            pltpu.VMEM((CC, KA), jnp.int32),
            pltpu.VMEM((CC, KA), jnp.int32),
            pltpu.VMEM((KA, Dh), jnp.float32),
            pltpu.VMEM((KA, Dh), jnp.float32),
            pltpu.VMEM_SHARED((N, Dh), jnp.float32),
            pltpu.SemaphoreType.DMA,
        ],
        compiler_params=pltpu.CompilerParams(use_tc_tiling_on_sc=False),
    )


DH1 = D1 // NC
DH2 = D2 // NC

_deg_call = _make_deg()
_agg1_call = _make_agg(DH1)
_agg2_call = _make_agg(DH2)


BN = 1000


def _prep_body(x_ref, ds_ref, dd_ref, w1_ref, y_ref, ns_ref, nd_ref):
    d_s = ds_ref[...]
    d_d = dd_ref[...]
    ns = lax.rsqrt(jnp.maximum(d_s[:, 0:1] + d_s[:, 1:2], 1.0))
    nd = lax.rsqrt(jnp.maximum(d_d[:, 0:1] + d_d[:, 1:2], 1.0))
    ns_ref[...] = ns
    nd_ref[...] = nd
    y = jnp.dot(x_ref[...] * ns, w1_ref[...],
                preferred_element_type=jnp.float32)
    y_ref[...] = jnp.stack([y[:, :DH1], y[:, DH1:]])


def _mid_body(p_ref, ns_ref, nd_ref, b1_ref, w2_ref, y2_ref):
    p = p_ref[...]
    agg = jnp.concatenate([p[0], p[1]], axis=1)
    h = jnp.maximum(agg * nd_ref[...] + b1_ref[...], 0.0)
    y2 = jnp.dot(h * ns_ref[...], w2_ref[...],
                 preferred_element_type=jnp.float32)
    y2_ref[...] = jnp.stack([y2[:, :DH2], y2[:, DH2:]])


def _fin_body(p_ref, nd_ref, b2_ref, o_ref):
    p = p_ref[...]
    o_ref[...] = jnp.concatenate([p[0], p[1]], axis=1) * nd_ref[...] + b2_ref[...]


def _prep(x, dsrcT, ddstT, W1):
    return pl.pallas_call(
        _prep_body,
        grid=(N // BN,),
        in_specs=[
            pl.BlockSpec((BN, D1), lambda i: (i, 0)),
            pl.BlockSpec((BN, 2), lambda i: (i, 0)),
            pl.BlockSpec((BN, 2), lambda i: (i, 0)),
            pl.BlockSpec((D1, D1), lambda i: (0, 0)),
        ],
        out_specs=[
            pl.BlockSpec((NC, BN, DH1), lambda i: (0, i, 0)),
            pl.BlockSpec((BN, 1), lambda i: (i, 0)),
            pl.BlockSpec((BN, 1), lambda i: (i, 0)),
        ],
        out_shape=[
            jax.ShapeDtypeStruct((NC, N, DH1), jnp.float32),
            jax.ShapeDtypeStruct((N, 1), jnp.float32),
            jax.ShapeDtypeStruct((N, 1), jnp.float32),
        ],
    )(x, dsrcT, ddstT, W1)


def _mid(agg1p, nsT, ndT, b1, W2):
    return pl.pallas_call(
        _mid_body,
        grid=(N // BN,),
        in_specs=[
            pl.BlockSpec((NC, BN, DH1), lambda i: (0, i, 0)),
            pl.BlockSpec((BN, 1), lambda i: (i, 0)),
            pl.BlockSpec((BN, 1), lambda i: (i, 0)),
            pl.BlockSpec((1, D1), lambda i: (0, 0)),
            pl.BlockSpec((D1, D2), lambda i: (0, 0)),
        ],
        out_specs=pl.BlockSpec((NC, BN, DH2), lambda i: (0, i, 0)),
        out_shape=jax.ShapeDtypeStruct((NC, N, DH2), jnp.float32),
    )(agg1p, nsT, ndT, b1, W2)


def _fin(agg2p, ndT, b2):
    return pl.pallas_call(
        _fin_body,
        grid=(N // BN,),
        in_specs=[
            pl.BlockSpec((NC, BN, DH2), lambda i: (0, i, 0)),
            pl.BlockSpec((BN, 1), lambda i: (i, 0)),
            pl.BlockSpec((1, D2), lambda i: (0, 0)),
        ],
        out_specs=pl.BlockSpec((BN, D2), lambda i: (i, 0)),
        out_shape=jax.ShapeDtypeStruct((N, D2), jnp.float32),
    )(agg2p, ndT, b2)


def kernel(x, edge_index, W1, b1, W2, b2):
    src32 = edge_index[0].astype(jnp.int32)
    dst32 = edge_index[1].astype(jnp.int32)
    src_d = src32.reshape(NW, C, K)
    dst_d = dst32.reshape(NW, C, K)
    src_a = src32.reshape(NS, CA, KA)
    dst_a = dst32.reshape(NS, CA, KA)
    ones_k = jnp.ones((K,), jnp.float32)
    zeros_n = jnp.zeros((N,), jnp.float32)

    degs = _deg_call(src_d, dst_d, ones_k, zeros_n).reshape(NC, 2, N)
    dsrcT = degs[:, 0, :].T
    ddstT = degs[:, 1, :].T

    y1, nsT, ndT = _prep(x, dsrcT, ddstT, W1)
    agg1h = _agg1_call(y1, src_a, dst_a)
    y2 = _mid(agg1h, nsT, ndT, b1.reshape(1, D1), W2)
    agg2h = _agg2_call(y2, src_a, dst_a)
    return _fin(agg2h, ndT, b2.reshape(1, D2))
